# baseline (speedup 1.0000x reference)
import functools

import jax
import jax.numpy as jnp
from jax import lax
from jax.experimental import pallas as pl
from jax.experimental.pallas import tpu as pltpu
from jax.experimental.pallas import tpu_sc as plsc

NC = 2
NS = 16
NW = NC * NS
L = 16
TW = 256


def _vgather(v, idx):
    dn = lax.GatherDimensionNumbers(
        offset_dims=(), collapsed_slice_dims=(0,), start_index_map=(0,))
    return lax.gather(v, idx.reshape(L, 1), dn, (1,),
                      mode=lax.GatherScatterMode.PROMISE_IN_BOUNDS)


def _repack_body(x_ref, out_ref):
    hb, D, W = x_ref.shape
    for h in range(hb):
        out_ref[pl.ds(h * W, W), :D] = x_ref[h].T
    out_ref[:, D:] = jnp.zeros((hb * W, TW - D), jnp.float32)


def _tc_repack(fm_t):
    BH, D, W = fm_t.shape
    HB = 8
    return pl.pallas_call(
        _repack_body,
        grid=(BH // HB,),
        in_specs=[pl.BlockSpec((HB, D, W), lambda i: (i, 0, 0))],
        out_specs=pl.BlockSpec((HB * W, TW), lambda i: (i, 0)),
        out_shape=jax.ShapeDtypeStruct((BH * W, TW), jnp.float32),
    )(fm_t)


def _sc_sample(verts_flat, table, B, N, H, W, D):
    BN = B * N
    vpw = BN // NW
    n_iters = vpw // L
    VCH = 32
    n_chunks = vpw // VCH

    mesh = plsc.VectorSubcoreMesh(
        core_axis_name="c", subcore_axis_name="s", num_cores=NC,
        num_subcores=NS)

    @functools.partial(
        pl.kernel,
        out_type=jax.ShapeDtypeStruct((BN, D), jnp.float32),
        mesh=mesh,
        scratch_types=[
            pltpu.VMEM((vpw * 2,), jnp.float32),
            pltpu.VMEM((4, vpw), jnp.int32),
            pltpu.VMEM((4, vpw), jnp.float32),
            pltpu.VMEM((2, 4, VCH, TW), jnp.float32),
            pltpu.VMEM((2, VCH, D), jnp.float32),
            pltpu.SemaphoreType.DMA,
            pltpu.SemaphoreType.DMA,
        ],
    )
    def k(verts_hbm, table_hbm, out_hbm, verts_v, idx_v, wgt_v, rows_v,
          feats_v, sem, sem_out):
        wid = lax.axis_index("s") * NC + lax.axis_index("c")
        vbase = wid * vpw
        base_row = (vbase // N) * (H * W)

        pltpu.sync_copy(verts_hbm.at[pl.ds(vbase * 2, vpw * 2)], verts_v)

        lane = lax.iota(jnp.int32, L)
        ia = (2 * lane) % L
        sel = lane < 8

        def idx_body(i, _):
            off = pl.multiple_of(i * (2 * L), 2 * L)
            va = verts_v[pl.ds(off, L)]
            vb = verts_v[pl.ds(off + L, L)]
            vy = jnp.where(sel, _vgather(va, ia), _vgather(vb, ia))
            vx = jnp.where(sel, _vgather(va, ia + 1), _vgather(vb, ia + 1))
            y = (vy + 1.0) * ((H - 1) * 0.5)
            x = (vx + 1.0) * ((W - 1) * 0.5)
            y = jnp.clip(y, 0.0, float(H - 1))
            x = jnp.clip(x, 0.0, float(W - 1))
            y0 = jnp.minimum(y.astype(jnp.int32), H - 2)
            x0 = jnp.minimum(x.astype(jnp.int32), W - 2)
            fy = y - y0.astype(jnp.float32)
            fx = x - x0.astype(jnp.float32)
            r00 = base_row + y0 * W + x0
            voff = pl.multiple_of(i * L, L)
            vsl = pl.ds(voff, L)
            idx_v[0, vsl] = r00
            idx_v[1, vsl] = r00 + 1
            idx_v[2, vsl] = r00 + W
            idx_v[3, vsl] = r00 + W + 1
            gy = 1.0 - fy
            gx = 1.0 - fx
            wgt_v[0, vsl] = gy * gx
            wgt_v[1, vsl] = gy * fx
            wgt_v[2, vsl] = fy * gx
            wgt_v[3, vsl] = fy * fx
            return 0

        lax.fori_loop(0, n_iters, idx_body, 0)

        def issue_gathers(g, buf):
            goff = pl.multiple_of(g * VCH, VCH)
            for kk in range(4):
                pltpu.async_copy(
                    table_hbm.at[idx_v.at[kk, pl.ds(goff, VCH)]],
                    rows_v.at[buf, kk], sem)

        def drain_gathers(buf):
            for kk in range(4):
                pltpu.make_async_copy(
                    table_hbm.at[idx_v.at[kk, pl.ds(0, VCH)]],
                    rows_v.at[buf, kk], sem).wait()

        def combine(g, buf):
            goff = pl.multiple_of(g * VCH, VCH)

            def group_body(q, _):
                qoff = pl.multiple_of(q * L, L)
                w0 = wgt_v[0, pl.ds(goff + qoff, L)]
                w1 = wgt_v[1, pl.ds(goff + qoff, L)]
                w2 = wgt_v[2, pl.ds(goff + qoff, L)]
                w3 = wgt_v[3, pl.ds(goff + qoff, L)]
                for j in range(L):
                    jv = jnp.full((L,), j, jnp.int32)
                    b0 = _vgather(w0, jv)
                    b1 = _vgather(w1, jv)
                    b2 = _vgather(w2, jv)
                    b3 = _vgather(w3, jv)
                    v = qoff + j
                    for s in range(D // L):
                        sl = pl.ds(s * L, L)
                        acc = b0 * rows_v[buf, 0, v, sl]
                        acc += b1 * rows_v[buf, 1, v, sl]
                        acc += b2 * rows_v[buf, 2, v, sl]
                        acc += b3 * rows_v[buf, 3, v, sl]
                        feats_v[buf, v, sl] = acc
                return 0

            lax.fori_loop(0, VCH // L, group_body, 0)

        def out_copy(g, buf):
            obase = pl.multiple_of(vbase + g * VCH, VCH)
            return pltpu.make_async_copy(
                feats_v.at[buf], out_hbm.at[pl.ds(obase, VCH)], sem_out)

        issue_gathers(0, 0)

        def ring_body(gg, _):
            for half in range(2):
                g = 2 * gg + half

                @pl.when(g + 1 < n_chunks)
                def _():
                    issue_gathers(g + 1, 1 - half)

                drain_gathers(half)

                @pl.when(g >= 2)
                def _():
                    out_copy(g - 2, half).wait()

                combine(g, half)
                out_copy(g, half).start()
            return 0

        lax.fori_loop(0, n_chunks // 2, ring_body, 0)
        out_copy(n_chunks - 2, 0).wait()
        out_copy(n_chunks - 1, 1).wait()

    return k(verts_flat, table)


def _mlp_body(x_ref, w1_ref, b1_ref, w2_ref, out_ref):
    h = jnp.dot(x_ref[...], w1_ref[...], preferred_element_type=jnp.float32)
    h = jnp.maximum(h + b1_ref[...], 0.0)
    out_ref[...] = jnp.dot(h, w2_ref[...], preferred_element_type=jnp.float32)


def _tc_mlp(feats, W1m, b1, W2m):
    BN, D = feats.shape
    DH = W1m.shape[1]
    DO = W2m.shape[1]
    BLK = 2048
    grid = (BN // BLK,)
    return pl.pallas_call(
        _mlp_body,
        grid=grid,
        in_specs=[
            pl.BlockSpec((BLK, D), lambda i: (i, 0)),
            pl.BlockSpec((D, DH), lambda i: (0, 0)),
            pl.BlockSpec((1, DH), lambda i: (0, 0)),
            pl.BlockSpec((DH, DO), lambda i: (0, 0)),
        ],
        out_specs=pl.BlockSpec((BLK, DO), lambda i: (i, 0)),
        out_shape=jax.ShapeDtypeStruct((BN, DO), jnp.float32),
    )(feats, W1m, b1.reshape(1, DH), W2m)


def kernel(vertices, feature_map, W1, b1, W2):
    B, N, _ = vertices.shape
    _, H, W, D = feature_map.shape
    fm_t = jnp.transpose(feature_map, (0, 1, 3, 2)).reshape(B * H, D, W)
    verts_flat = vertices.reshape(B * N * 2)
    STAGES = 4
    bs = B // STAGES
    feats = []
    for s in range(STAGES):
        fm_s = lax.slice_in_dim(fm_t, s * bs * H, (s + 1) * bs * H, axis=0)
        table_s = _tc_repack(fm_s)
        verts_s = lax.slice_in_dim(verts_flat, s * bs * N * 2,
                                   (s + 1) * bs * N * 2, axis=0)
        feats.append(_sc_sample(verts_s, table_s, bs, N, H, W, D))
    outs = [_tc_mlp(f, W1[0], b1, W2[0]) for f in feats]
    out = jnp.concatenate(outs, axis=0)
    return out.reshape(B, N, 2)


# speedup vs baseline: 1.5107x; 1.5107x over previous
import functools

import jax
import jax.numpy as jnp
from jax import lax
from jax.experimental import pallas as pl
from jax.experimental.pallas import tpu as pltpu
from jax.experimental.pallas import tpu_sc as plsc

NC = 2
NS = 16
NW = NC * NS
L = 16
TW = 256


def _vgather(v, idx):
    dn = lax.GatherDimensionNumbers(
        offset_dims=(), collapsed_slice_dims=(0,), start_index_map=(0,))
    return lax.gather(v, idx.reshape(L, 1), dn, (1,),
                      mode=lax.GatherScatterMode.PROMISE_IN_BOUNDS)


def _repack_body(x_ref, out_ref):
    hb, D, W = x_ref.shape
    for h in range(hb):
        out_ref[pl.ds(h * W, W), :D] = x_ref[h].T
    out_ref[:, D:] = jnp.zeros((hb * W, TW - D), jnp.float32)


def _tc_repack(fm_t, bh0, nbh):
    _, D, W = fm_t.shape
    HB = 8
    return pl.pallas_call(
        _repack_body,
        grid=(nbh // HB,),
        in_specs=[pl.BlockSpec((HB, D, W), lambda i: (i + bh0 // HB, 0, 0))],
        out_specs=pl.BlockSpec((HB * W, TW), lambda i: (i, 0)),
        out_shape=jax.ShapeDtypeStruct((nbh * W, TW), jnp.float32),
    )(fm_t)


def _sc_sample(verts_flat, table, B, N, H, W, D, v0=0):
    BN = B * N
    vpw = BN // NW
    n_iters = vpw // L
    VCH = 32
    n_chunks = vpw // VCH

    mesh = plsc.VectorSubcoreMesh(
        core_axis_name="c", subcore_axis_name="s", num_cores=NC,
        num_subcores=NS)

    @functools.partial(
        pl.kernel,
        out_type=jax.ShapeDtypeStruct((BN, D), jnp.float32),
        mesh=mesh,
        scratch_types=[
            pltpu.VMEM((vpw * 2,), jnp.float32),
            pltpu.VMEM((4, vpw), jnp.int32),
            pltpu.VMEM((4, vpw), jnp.float32),
            pltpu.VMEM((2, 4, VCH, TW), jnp.float32),
            pltpu.VMEM((2, VCH, D), jnp.float32),
            pltpu.SemaphoreType.DMA,
            pltpu.SemaphoreType.DMA,
        ],
    )
    def k(verts_hbm, table_hbm, out_hbm, verts_v, idx_v, wgt_v, rows_v,
          feats_v, sem, sem_out):
        wid = lax.axis_index("s") * NC + lax.axis_index("c")
        vbase = wid * vpw
        base_row = (vbase // N) * (H * W)

        pltpu.sync_copy(verts_hbm.at[pl.ds((v0 + vbase) * 2, vpw * 2)],
                        verts_v)

        lane = lax.iota(jnp.int32, L)
        ia = (2 * lane) % L
        sel = lane < 8

        def idx_body(i, _):
            off = pl.multiple_of(i * (2 * L), 2 * L)
            va = verts_v[pl.ds(off, L)]
            vb = verts_v[pl.ds(off + L, L)]
            vy = jnp.where(sel, _vgather(va, ia), _vgather(vb, ia))
            vx = jnp.where(sel, _vgather(va, ia + 1), _vgather(vb, ia + 1))
            y = (vy + 1.0) * ((H - 1) * 0.5)
            x = (vx + 1.0) * ((W - 1) * 0.5)
            y = jnp.clip(y, 0.0, float(H - 1))
            x = jnp.clip(x, 0.0, float(W - 1))
            y0 = jnp.minimum(y.astype(jnp.int32), H - 2)
            x0 = jnp.minimum(x.astype(jnp.int32), W - 2)
            fy = y - y0.astype(jnp.float32)
            fx = x - x0.astype(jnp.float32)
            r00 = base_row + y0 * W + x0
            voff = pl.multiple_of(i * L, L)
            vsl = pl.ds(voff, L)
            idx_v[0, vsl] = r00
            idx_v[1, vsl] = r00 + 1
            idx_v[2, vsl] = r00 + W
            idx_v[3, vsl] = r00 + W + 1
            gy = 1.0 - fy
            gx = 1.0 - fx
            wgt_v[0, vsl] = gy * gx
            wgt_v[1, vsl] = gy * fx
            wgt_v[2, vsl] = fy * gx
            wgt_v[3, vsl] = fy * fx
            return 0

        lax.fori_loop(0, n_iters, idx_body, 0)

        def issue_gathers(g, buf):
            goff = pl.multiple_of(g * VCH, VCH)
            for kk in range(4):
                pltpu.async_copy(
                    table_hbm.at[idx_v.at[kk, pl.ds(goff, VCH)]],
                    rows_v.at[buf, kk], sem)

        def drain_gathers(buf):
            for kk in range(4):
                pltpu.make_async_copy(
                    table_hbm.at[idx_v.at[kk, pl.ds(0, VCH)]],
                    rows_v.at[buf, kk], sem).wait()

        def combine(g, buf):
            goff = pl.multiple_of(g * VCH, VCH)

            def group_body(q, _):
                qoff = pl.multiple_of(q * L, L)
                w0 = wgt_v[0, pl.ds(goff + qoff, L)]
                w1 = wgt_v[1, pl.ds(goff + qoff, L)]
                w2 = wgt_v[2, pl.ds(goff + qoff, L)]
                w3 = wgt_v[3, pl.ds(goff + qoff, L)]
                for j in range(L):
                    jv = jnp.full((L,), j, jnp.int32)
                    b0 = _vgather(w0, jv)
                    b1 = _vgather(w1, jv)
                    b2 = _vgather(w2, jv)
                    b3 = _vgather(w3, jv)
                    v = qoff + j
                    for s in range(D // L):
                        sl = pl.ds(s * L, L)
                        acc = b0 * rows_v[buf, 0, v, sl]
                        acc += b1 * rows_v[buf, 1, v, sl]
                        acc += b2 * rows_v[buf, 2, v, sl]
                        acc += b3 * rows_v[buf, 3, v, sl]
                        feats_v[buf, v, sl] = acc
                return 0

            lax.fori_loop(0, VCH // L, group_body, 0)

        def out_copy(g, buf):
            obase = pl.multiple_of(vbase + g * VCH, VCH)
            return pltpu.make_async_copy(
                feats_v.at[buf], out_hbm.at[pl.ds(obase, VCH)], sem_out)

        issue_gathers(0, 0)

        def ring_body(gg, _):
            for half in range(2):
                g = 2 * gg + half

                @pl.when(g + 1 < n_chunks)
                def _():
                    issue_gathers(g + 1, 1 - half)

                drain_gathers(half)

                @pl.when(g >= 2)
                def _():
                    out_copy(g - 2, half).wait()

                combine(g, half)
                out_copy(g, half).start()
            return 0

        lax.fori_loop(0, n_chunks // 2, ring_body, 0)
        out_copy(n_chunks - 2, 0).wait()
        out_copy(n_chunks - 1, 1).wait()

    return k(verts_flat, table)


def _mlp_body(x_ref, w1_ref, b1_ref, w2_ref, out_ref):
    h = jnp.dot(x_ref[...], w1_ref[...], preferred_element_type=jnp.float32)
    h = jnp.maximum(h + b1_ref[...], 0.0)
    out_ref[...] = jnp.dot(h, w2_ref[...], preferred_element_type=jnp.float32)


def _tc_mlp(feats, W1m, b1, W2m):
    BN, D = feats.shape
    DH = W1m.shape[1]
    DO = W2m.shape[1]
    BLK = 2048
    grid = (BN // BLK,)
    return pl.pallas_call(
        _mlp_body,
        grid=grid,
        in_specs=[
            pl.BlockSpec((BLK, D), lambda i: (i, 0)),
            pl.BlockSpec((D, DH), lambda i: (0, 0)),
            pl.BlockSpec((1, DH), lambda i: (0, 0)),
            pl.BlockSpec((DH, DO), lambda i: (0, 0)),
        ],
        out_specs=pl.BlockSpec((BLK, DO), lambda i: (i, 0)),
        out_shape=jax.ShapeDtypeStruct((BN, DO), jnp.float32),
    )(feats, W1m, b1.reshape(1, DH), W2m)


def kernel(vertices, feature_map, W1, b1, W2):
    B, N, _ = vertices.shape
    _, H, W, D = feature_map.shape
    fm_t = jnp.transpose(feature_map, (0, 1, 3, 2)).reshape(B * H, D, W)
    verts_flat = vertices.reshape(B * N * 2)
    STAGES = 4
    bs = B // STAGES
    feats = []
    for s in range(STAGES):
        table_s = _tc_repack(fm_t, s * bs * H, bs * H)
        feats.append(_sc_sample(verts_flat, table_s, bs, N, H, W, D,
                                v0=s * bs * N))
    outs = [_tc_mlp(f, W1[0], b1, W2[0]) for f in feats]
    out = jnp.concatenate(outs, axis=0)
    return out.reshape(B, N, 2)


# speedup vs baseline: 1.5831x; 1.0479x over previous
import functools

import jax
import jax.numpy as jnp
from jax import lax
from jax.experimental import pallas as pl
from jax.experimental.pallas import tpu as pltpu
from jax.experimental.pallas import tpu_sc as plsc

NC = 2
NS = 16
NW = NC * NS
L = 16
TW = 256


def _vgather(v, idx):
    dn = lax.GatherDimensionNumbers(
        offset_dims=(), collapsed_slice_dims=(0,), start_index_map=(0,))
    return lax.gather(v, idx.reshape(L, 1), dn, (1,),
                      mode=lax.GatherScatterMode.PROMISE_IN_BOUNDS)


def _repack_body(x_ref, out_ref):
    hb, D, W = x_ref.shape
    for h in range(hb):
        out_ref[pl.ds(h * W, W), :D] = x_ref[h].T
    out_ref[:, D:] = jnp.zeros((hb * W, TW - D), jnp.float32)


def _tc_repack(fm_t, bh0, nbh):
    _, D, W = fm_t.shape
    HB = 8
    return pl.pallas_call(
        _repack_body,
        grid=(nbh // HB,),
        in_specs=[pl.BlockSpec((HB, D, W), lambda i: (i + bh0 // HB, 0, 0))],
        out_specs=pl.BlockSpec((HB * W, TW), lambda i: (i, 0)),
        out_shape=jax.ShapeDtypeStruct((nbh * W, TW), jnp.float32),
    )(fm_t)


def _sc_sample(verts_flat, table, B, N, H, W, D, v0=0):
    BN = B * N
    vpw = BN // NW
    n_iters = vpw // L
    VCH = 32
    n_chunks = vpw // VCH

    mesh = plsc.VectorSubcoreMesh(
        core_axis_name="c", subcore_axis_name="s", num_cores=NC,
        num_subcores=NS)

    @functools.partial(
        pl.kernel,
        out_type=jax.ShapeDtypeStruct((BN, D), jnp.float32),
        mesh=mesh,
        scratch_types=[
            pltpu.VMEM((2, vpw), jnp.float32),
            pltpu.VMEM((4, vpw), jnp.int32),
            pltpu.VMEM((4, vpw), jnp.float32),
            pltpu.VMEM((2, 4, VCH, TW), jnp.float32),
            pltpu.VMEM((2, VCH, D), jnp.float32),
            pltpu.SemaphoreType.DMA,
            pltpu.SemaphoreType.DMA,
        ],
    )
    def k(verts_hbm, table_hbm, out_hbm, verts_v, idx_v, wgt_v, rows_v,
          feats_v, sem, sem_out):
        wid = lax.axis_index("s") * NC + lax.axis_index("c")
        vbase = wid * vpw
        base_row = (vbase // N) * (H * W)

        bg = (v0 + vbase) // N
        n0 = (v0 + vbase) % N
        pltpu.sync_copy(verts_hbm.at[pl.ds(bg * 2 * N + n0, vpw)],
                        verts_v.at[0])
        pltpu.sync_copy(verts_hbm.at[pl.ds(bg * 2 * N + N + n0, vpw)],
                        verts_v.at[1])

        def idx_body(i, _):
            voff0 = pl.multiple_of(i * L, L)
            vy = verts_v[0, pl.ds(voff0, L)]
            vx = verts_v[1, pl.ds(voff0, L)]
            y = (vy + 1.0) * ((H - 1) * 0.5)
            x = (vx + 1.0) * ((W - 1) * 0.5)
            y = jnp.clip(y, 0.0, float(H - 1))
            x = jnp.clip(x, 0.0, float(W - 1))
            y0 = jnp.minimum(y.astype(jnp.int32), H - 2)
            x0 = jnp.minimum(x.astype(jnp.int32), W - 2)
            fy = y - y0.astype(jnp.float32)
            fx = x - x0.astype(jnp.float32)
            r00 = base_row + y0 * W + x0
            voff = pl.multiple_of(i * L, L)
            vsl = pl.ds(voff, L)
            idx_v[0, vsl] = r00
            idx_v[1, vsl] = r00 + 1
            idx_v[2, vsl] = r00 + W
            idx_v[3, vsl] = r00 + W + 1
            gy = 1.0 - fy
            gx = 1.0 - fx
            wgt_v[0, vsl] = gy * gx
            wgt_v[1, vsl] = gy * fx
            wgt_v[2, vsl] = fy * gx
            wgt_v[3, vsl] = fy * fx
            return 0

        lax.fori_loop(0, n_iters, idx_body, 0)

        def issue_gathers(g, buf):
            goff = pl.multiple_of(g * VCH, VCH)
            for kk in range(4):
                pltpu.async_copy(
                    table_hbm.at[idx_v.at[kk, pl.ds(goff, VCH)]],
                    rows_v.at[buf, kk], sem)

        def drain_gathers(buf):
            for kk in range(4):
                pltpu.make_async_copy(
                    table_hbm.at[idx_v.at[kk, pl.ds(0, VCH)]],
                    rows_v.at[buf, kk], sem).wait()

        def combine(g, buf):
            goff = pl.multiple_of(g * VCH, VCH)

            def group_body(q, _):
                qoff = pl.multiple_of(q * L, L)
                w0 = wgt_v[0, pl.ds(goff + qoff, L)]
                w1 = wgt_v[1, pl.ds(goff + qoff, L)]
                w2 = wgt_v[2, pl.ds(goff + qoff, L)]
                w3 = wgt_v[3, pl.ds(goff + qoff, L)]
                for j in range(L):
                    jv = jnp.full((L,), j, jnp.int32)
                    b0 = _vgather(w0, jv)
                    b1 = _vgather(w1, jv)
                    b2 = _vgather(w2, jv)
                    b3 = _vgather(w3, jv)
                    v = qoff + j
                    for s in range(D // L):
                        sl = pl.ds(s * L, L)
                        acc = b0 * rows_v[buf, 0, v, sl]
                        acc += b1 * rows_v[buf, 1, v, sl]
                        acc += b2 * rows_v[buf, 2, v, sl]
                        acc += b3 * rows_v[buf, 3, v, sl]
                        feats_v[buf, v, sl] = acc
                return 0

            lax.fori_loop(0, VCH // L, group_body, 0)

        def out_copy(g, buf):
            obase = pl.multiple_of(vbase + g * VCH, VCH)
            return pltpu.make_async_copy(
                feats_v.at[buf], out_hbm.at[pl.ds(obase, VCH)], sem_out)

        issue_gathers(0, 0)

        def ring_body(gg, _):
            for half in range(2):
                g = 2 * gg + half

                @pl.when(g + 1 < n_chunks)
                def _():
                    issue_gathers(g + 1, 1 - half)

                drain_gathers(half)

                @pl.when(g >= 2)
                def _():
                    out_copy(g - 2, half).wait()

                combine(g, half)
                out_copy(g, half).start()
            return 0

        lax.fori_loop(0, n_chunks // 2, ring_body, 0)
        out_copy(n_chunks - 2, 0).wait()
        out_copy(n_chunks - 1, 1).wait()

    return k(verts_flat, table)


def _mlp_body(x_ref, w1_ref, b1_ref, w2_ref, out_ref):
    h = jnp.dot(x_ref[...], w1_ref[...], preferred_element_type=jnp.float32)
    h = jnp.maximum(h + b1_ref[...], 0.0)
    out_ref[...] = jnp.dot(h, w2_ref[...], preferred_element_type=jnp.float32)


def _tc_mlp(feats, W1m, b1, W2m):
    BN, D = feats.shape
    DH = W1m.shape[1]
    DO = W2m.shape[1]
    BLK = 2048
    grid = (BN // BLK,)
    return pl.pallas_call(
        _mlp_body,
        grid=grid,
        in_specs=[
            pl.BlockSpec((BLK, D), lambda i: (i, 0)),
            pl.BlockSpec((D, DH), lambda i: (0, 0)),
            pl.BlockSpec((1, DH), lambda i: (0, 0)),
            pl.BlockSpec((DH, DO), lambda i: (0, 0)),
        ],
        out_specs=pl.BlockSpec((BLK, DO), lambda i: (i, 0)),
        out_shape=jax.ShapeDtypeStruct((BN, DO), jnp.float32),
    )(feats, W1m, b1.reshape(1, DH), W2m)


def kernel(vertices, feature_map, W1, b1, W2):
    B, N, _ = vertices.shape
    _, H, W, D = feature_map.shape
    fm_t = jnp.transpose(feature_map, (0, 1, 3, 2)).reshape(B * H, D, W)
    verts_flat = jnp.transpose(vertices, (0, 2, 1)).reshape(B * 2 * N)
    STAGES = 4
    bs = B // STAGES
    feats = []
    for s in range(STAGES):
        table_s = _tc_repack(fm_t, s * bs * H, bs * H)
        feats.append(_sc_sample(verts_flat, table_s, bs, N, H, W, D,
                                v0=s * bs * N))
    outs = [_tc_mlp(f, W1[0], b1, W2[0]) for f in feats]
    out = jnp.concatenate(outs, axis=0)
    return out.reshape(B, N, 2)


# speedup vs baseline: 1.5866x; 1.0022x over previous
import functools

import jax
import jax.numpy as jnp
from jax import lax
from jax.experimental import pallas as pl
from jax.experimental.pallas import tpu as pltpu
from jax.experimental.pallas import tpu_sc as plsc

NC = 2
NS = 16
NW = NC * NS
L = 16
TW = 256
TP = 128
HC = 96


def _vgather(v, idx):
    dn = lax.GatherDimensionNumbers(
        offset_dims=(), collapsed_slice_dims=(0,), start_index_map=(0,))
    return lax.gather(v, idx.reshape(L, 1), dn, (1,),
                      mode=lax.GatherScatterMode.PROMISE_IN_BOUNDS)


def _repack_body(x_ref, out_ref):
    hb, D, W = x_ref.shape
    for h in range(hb):
        t = x_ref[h].T
        rb = lax.shift_right_logical(
            lax.bitcast_convert_type(t, jnp.int32) + 0x8000, 16)
        packed = rb[:, :HC] | lax.shift_left(rb[:, HC:2 * HC], 16)
        out_ref[pl.ds(h * W, W), :HC] = packed
        out_ref[pl.ds(h * W, W), HC:] = jnp.zeros((W, TP - HC), jnp.int32)


def _tc_repack(fm_t, bh0, nbh):
    _, D, W = fm_t.shape
    HB = 8
    return pl.pallas_call(
        _repack_body,
        grid=(nbh // HB,),
        in_specs=[pl.BlockSpec((HB, D, W), lambda i: (i + bh0 // HB, 0, 0))],
        out_specs=pl.BlockSpec((HB * W, TP), lambda i: (i, 0)),
        out_shape=jax.ShapeDtypeStruct((nbh * W, TP), jnp.int32),
    )(fm_t)


def _sc_sample(verts_flat, table, B, N, H, W, D, v0=0):
    BN = B * N
    vpw = BN // NW
    n_iters = vpw // L
    VCH = 32
    n_chunks = vpw // VCH

    mesh = plsc.VectorSubcoreMesh(
        core_axis_name="c", subcore_axis_name="s", num_cores=NC,
        num_subcores=NS)

    @functools.partial(
        pl.kernel,
        out_type=(jax.ShapeDtypeStruct((4, BN, TP), jnp.int32),
                  jax.ShapeDtypeStruct((4, BN), jnp.float32)),
        mesh=mesh,
        scratch_types=[
            pltpu.VMEM((2, vpw), jnp.float32),
            pltpu.VMEM((4, vpw), jnp.int32),
            pltpu.VMEM((4, vpw), jnp.float32),
            pltpu.VMEM((2, 4, VCH, TP), jnp.int32),
            pltpu.SemaphoreType.DMA,
            pltpu.SemaphoreType.DMA,
        ],
    )
    def k(verts_hbm, table_hbm, rows_hbm, wgt_hbm, verts_v, idx_v, wgt_v,
          rows_v, sem, sem_out):
        wid = lax.axis_index("s") * NC + lax.axis_index("c")
        vbase = wid * vpw
        base_row = (vbase // N) * (H * W)

        bg = (v0 + vbase) // N
        n0 = (v0 + vbase) % N
        pltpu.sync_copy(verts_hbm.at[pl.ds(bg * 2 * N + n0, vpw)],
                        verts_v.at[0])
        pltpu.sync_copy(verts_hbm.at[pl.ds(bg * 2 * N + N + n0, vpw)],
                        verts_v.at[1])

        def idx_body(i, _):
            voff0 = pl.multiple_of(i * L, L)
            vy = verts_v[0, pl.ds(voff0, L)]
            vx = verts_v[1, pl.ds(voff0, L)]
            y = (vy + 1.0) * ((H - 1) * 0.5)
            x = (vx + 1.0) * ((W - 1) * 0.5)
            y = jnp.clip(y, 0.0, float(H - 1))
            x = jnp.clip(x, 0.0, float(W - 1))
            y0 = jnp.minimum(y.astype(jnp.int32), H - 2)
            x0 = jnp.minimum(x.astype(jnp.int32), W - 2)
            fy = y - y0.astype(jnp.float32)
            fx = x - x0.astype(jnp.float32)
            r00 = base_row + y0 * W + x0
            vsl = pl.ds(voff0, L)
            idx_v[0, vsl] = r00
            idx_v[1, vsl] = r00 + 1
            idx_v[2, vsl] = r00 + W
            idx_v[3, vsl] = r00 + W + 1
            gy = 1.0 - fy
            gx = 1.0 - fx
            wgt_v[0, vsl] = gy * gx
            wgt_v[1, vsl] = gy * fx
            wgt_v[2, vsl] = fy * gx
            wgt_v[3, vsl] = fy * fx
            return 0

        lax.fori_loop(0, n_iters, idx_body, 0)

        for kk in range(4):
            pltpu.sync_copy(wgt_v.at[kk], wgt_hbm.at[kk, pl.ds(vbase, vpw)])

        def issue_gathers(g, buf):
            goff = pl.multiple_of(g * VCH, VCH)
            for kk in range(4):
                pltpu.async_copy(
                    table_hbm.at[idx_v.at[kk, pl.ds(goff, VCH)]],
                    rows_v.at[buf, kk], sem)

        def drain_gathers(buf):
            for kk in range(4):
                pltpu.make_async_copy(
                    table_hbm.at[idx_v.at[kk, pl.ds(0, VCH)]],
                    rows_v.at[buf, kk], sem).wait()

        def out_copy(g, buf, kk):
            obase = pl.multiple_of(vbase + g * VCH, VCH)
            return pltpu.make_async_copy(
                rows_v.at[buf, kk], rows_hbm.at[kk, pl.ds(obase, VCH)],
                sem_out)

        issue_gathers(0, 0)

        def ring_body(gg, _):
            for half in range(2):
                g = 2 * gg + half

                drain_gathers(half)

                @pl.when(g >= 2)
                def _():
                    for kk in range(4):
                        out_copy(g - 2, half, kk).wait()

                for kk in range(4):
                    out_copy(g, half, kk).start()

                @pl.when(g + 1 < n_chunks)
                def _():
                    issue_gathers(g + 1, 1 - half)
            return 0

        lax.fori_loop(0, n_chunks // 2, ring_body, 0)
        for kk in range(4):
            out_copy(n_chunks - 2, 0, kk).wait()
            out_copy(n_chunks - 1, 1, kk).wait()

    return k(verts_flat, table)


def _mlp_body(r_ref, w_ref, w1_ref, b1_ref, w2_ref, out_ref):
    BLK = r_ref.shape[1]
    feats = jnp.zeros((BLK, TW), jnp.float32)
    for kk in range(4):
        r = r_ref[kk]
        fl = lax.bitcast_convert_type(lax.shift_left(r, 16), jnp.float32)
        fh = lax.bitcast_convert_type(r & jnp.int32(-65536), jnp.float32)
        rf = jnp.concatenate([fl, fh], axis=1)
        feats = feats + w_ref[kk].reshape(BLK, 1) * rf
    h = jnp.dot(feats, w1_ref[...], preferred_element_type=jnp.float32)
    h = jnp.maximum(h + b1_ref[...], 0.0)
    out_ref[...] = jnp.dot(h, w2_ref[...], preferred_element_type=jnp.float32)


def _tc_mlp(rows, wgt, W1p, b1, W2m):
    _, BN, _ = rows.shape
    DH = W1p.shape[1]
    DO = W2m.shape[1]
    BLK = 1024
    grid = (BN // BLK,)
    return pl.pallas_call(
        _mlp_body,
        grid=grid,
        in_specs=[
            pl.BlockSpec((4, BLK, TP), lambda i: (0, i, 0)),
            pl.BlockSpec((4, BLK), lambda i: (0, i)),
            pl.BlockSpec((TW, DH), lambda i: (0, 0)),
            pl.BlockSpec((1, DH), lambda i: (0, 0)),
            pl.BlockSpec((DH, DO), lambda i: (0, 0)),
        ],
        out_specs=pl.BlockSpec((BLK, DO), lambda i: (i, 0)),
        out_shape=jax.ShapeDtypeStruct((BN, DO), jnp.float32),
    )(rows, wgt, W1p, b1.reshape(1, DH), W2m)


def kernel(vertices, feature_map, W1, b1, W2):
    B, N, _ = vertices.shape
    _, H, W, D = feature_map.shape
    fm_t = jnp.transpose(feature_map, (0, 1, 3, 2)).reshape(B * H, D, W)
    verts_flat = jnp.transpose(vertices, (0, 2, 1)).reshape(B * 2 * N)
    DH = W1.shape[2]
    zp = jnp.zeros((TP - HC, DH), jnp.float32)
    W1p = jnp.concatenate([W1[0][:HC], zp, W1[0][HC:D], zp], axis=0)
    STAGES = 4
    bs = B // STAGES
    samples = []
    for s in range(STAGES):
        table_s = _tc_repack(fm_t, s * bs * H, bs * H)
        samples.append(_sc_sample(verts_flat, table_s, bs, N, H, W, D,
                                  v0=s * bs * N))
    outs = [_tc_mlp(rows_s, wgt_s, W1p, b1, W2[0])
            for rows_s, wgt_s in samples]
    out = jnp.concatenate(outs, axis=0)
    return out.reshape(B, N, 2)
